# Initial kernel scaffold; baseline (speedup 1.0000x reference)
import jax
import jax.numpy as jnp
from jax.experimental import pallas as pl


def kernel(x_pa, ply, W, b):
    raise NotImplementedError("write your pallas kernel here")


# speedup vs baseline: 1.4049x; 1.4049x over previous
import functools

import jax
import jax.numpy as jnp
from jax import lax
from jax.experimental import pallas as pl
from jax.experimental.pallas import tpu as pltpu
from jax.experimental.pallas import tpu_sc as plsc

LINPUT = 256
COUNT = 10
BUCKET_SIZE = 6
BATCH = 16384

NC = 2
NS = 16
L = 16
NW = NC * NS
BPW = BATCH // NW
XCH = 256
NCHUNK = BPW // XCH
NJ = LINPUT // L


_GDN = lax.GatherDimensionNumbers(
    offset_dims=(), collapsed_slice_dims=(0,), start_index_map=(0,))


def _permute(v, idx):
    return lax.gather(v, idx[:, None], _GDN, (1,),
                      mode=lax.GatherScatterMode.PROMISE_IN_BOUNDS)


def _sc_body(x_hbm, ply_hbm, w_hbm, b_hbm, out_hbm,
             w_v, b_v, ply_v, c_v, xbuf, out_v):
    wid = lax.axis_index("s") * NC + lax.axis_index("c")
    base = wid * BPW

    pltpu.sync_copy(w_hbm, w_v)
    pltpu.sync_copy(b_hbm, b_v)
    pltpu.sync_copy(ply_hbm.at[pl.ds(base, BPW)], ply_v)

    lane = lax.iota(jnp.int32, L)
    bvec = b_v[pl.ds(0, L)]
    bs = [bvec[c0] for c0 in range(COUNT)]

    def chunk_body(ch, carry):
        pltpu.sync_copy(x_hbm.at[pl.ds(base + ch * XCH, XCH)], xbuf)

        def group_body(g, carry2):
            gs = ch * XCH + g * L
            cvec = lax.shift_right_logical(ply_v[pl.ds(gs, L)] * 10923, 16)
            accs = []
            for s in range(L):
                row = g * L + s
                woff = cvec[s] * LINPUT
                acc = xbuf[row, pl.ds(0, L)] * w_v[pl.ds(woff, L)]
                for jc in range(1, NJ):
                    acc = acc + xbuf[row, pl.ds(jc * L, L)] * w_v[pl.ds(woff + jc * L, L)]
                accs.append(acc)
            m = 1
            while len(accs) > 1:
                sel = (lane & m) == 0
                perm = lane ^ m
                nxt = []
                for j in range(len(accs) // 2):
                    a, c = accs[2 * j], accs[2 * j + 1]
                    ax = _permute(a, perm)
                    cx = _permute(c, perm)
                    nxt.append(jnp.where(sel, a + ax, c + cx))
                accs = nxt
                m *= 2
            badd = jnp.zeros((L,), jnp.float32)
            for c0 in range(COUNT):
                badd = jnp.where(cvec == c0, bs[c0], badd)
            out_v[pl.ds(gs, L)] = accs[0] + badd
            return carry2

        lax.fori_loop(0, XCH // L, group_body, 0)
        return carry

    lax.fori_loop(0, NCHUNK, chunk_body, 0)
    pltpu.sync_copy(out_v, out_hbm.at[pl.ds(base, BPW)])


@jax.jit
def _run(x_pa, ply, wf, bf):
    mesh = plsc.VectorSubcoreMesh(core_axis_name="c", subcore_axis_name="s")
    f = functools.partial(
        pl.kernel,
        out_type=jax.ShapeDtypeStruct((BATCH,), jnp.float32),
        mesh=mesh,
        scratch_types=[
            pltpu.VMEM((COUNT * LINPUT,), jnp.float32),
            pltpu.VMEM((L,), jnp.float32),
            pltpu.VMEM((BPW,), jnp.int32),
            pltpu.VMEM((BPW,), jnp.int32),
            pltpu.VMEM((XCH, LINPUT), jnp.float32),
            pltpu.VMEM((BPW,), jnp.float32),
        ],
    )(_sc_body)
    return f(x_pa, ply, wf, bf)


def kernel(x_pa, ply, W, b):
    wf = W.reshape(COUNT * LINPUT)
    bf = jnp.zeros((L,), jnp.float32).at[:COUNT].set(b.reshape(COUNT))
    out = _run(x_pa, ply, wf, bf)
    return out.reshape(BATCH, 1)
